# baseline (speedup 1.0000x reference)
import functools
import jax
from jax import lax
import jax.numpy as jnp
from jax.experimental import pallas as pl
from jax.experimental.pallas import tpu as pltpu
from jax.experimental.pallas import tpu_sc as plsc

B, N, IN_DIM, DEP = 8, 64, 128, 8
K = N * DEP
ROWS = B * N
NC, NS, L = 2, 16, 16
NW = NC * NS
RPW = ROWS // NW
WPB = N // RPW
NCH = N // L

_mesh = plsc.VectorSubcoreMesh(core_axis_name="c", subcore_axis_name="s")


def _sc_body(adj_hbm, w_hbm, asp_hbm, s_hbm, adjv, sv, wv, av):
    wid = lax.axis_index("s") * NC + lax.axis_index("c")
    b = wid // WPB
    a0 = (wid % WPB) * RPW
    pltpu.sync_copy(adj_hbm.at[b, pl.ds(a0, RPW)], adjv)
    pltpu.sync_copy(w_hbm.at[0, pl.ds(IN_DIM, L)], wv)
    pltpu.sync_copy(asp_hbm.at[b, pl.ds(a0, RPW)], av)
    lanes = lax.iota(jnp.int32, L)
    lanesf = lanes.astype(jnp.float32)
    w2vec = wv[...]
    zeros = jnp.zeros((L,), jnp.float32)
    onehot = [(lanesf == float(d)).astype(jnp.float32) for d in range(DEP)]

    def row_body(row, carry):
        rowvec = jnp.broadcast_to(row, (L,))
        aspf = (plsc.load_gather(av, [rowvec]) != 0).astype(jnp.float32)
        cacc = [zeros for _ in range(NCH)]
        vacc = [zeros for _ in range(NCH)]
        radd = zeros
        for d in range(DEP):
            w2d = w2vec[d]
            rdi = jnp.zeros((L,), jnp.int32)
            for ch in range(NCH):
                g = adjv[row, d, pl.ds(L * ch, L)]
                m = jnp.abs(g) > 0.0
                cacc[ch] = cacc[ch] + jnp.where(m, 1.0, 0.0)
                vacc[ch] = vacc[ch] + g * w2d
                rdi = rdi + plsc.all_reduce_population_count(m)
            radd = radd + rdi.astype(jnp.float32) * onehot[d]
        cacc[0] = cacc[0] + radd
        for ch in range(NCH):
            sv[row, pl.ds(ch * L, L)] = (cacc[ch] * aspf) * jnp.exp(vacc[ch])
        for ch in range(NCH, IN_DIM // L):
            sv[row, pl.ds(ch * L, L)] = zeros
        return carry

    lax.fori_loop(0, RPW, row_body, 0)
    pltpu.sync_copy(sv, s_hbm.at[b, pl.ds(a0, RPW)])


_sc_weights = functools.partial(
    pl.kernel,
    out_type=jax.ShapeDtypeStruct((B, N, IN_DIM), jnp.float32),
    mesh=_mesh,
    compiler_params=pltpu.CompilerParams(
        needs_layout_passes=False, skip_device_barrier=True),
    scratch_types=[
        pltpu.VMEM((RPW, DEP, N), jnp.float32),
        pltpu.VMEM((RPW, IN_DIM), jnp.float32),
        pltpu.VMEM((L,), jnp.float32),
        pltpu.VMEM((RPW,), jnp.int32),
    ],
)(_sc_body)

_CONTRACT_LAST = (((1,), (1,)), ((), ()))


def _tc_body(f_ref, s_ref, w_ref, out_ref):
    w1row = w_ref[:, :IN_DIM]
    hi = jax.lax.Precision.HIGHEST
    for b in range(B):
        f = f_ref[b]
        sb = s_ref[b][:, :N]
        urow = lax.dot_general(w1row, f, _CONTRACT_LAST, precision=hi)
        st = sb * jnp.exp(urow)
        num = jnp.dot(st, f, precision=hi)
        den = jnp.sum(st, axis=1, keepdims=True)
        out_ref[b] = jnp.where(den > 0.0, num / den, f)


def kernel(features, aspect_onehot, adj_matrix, W):
    adj_t = adj_matrix.transpose(0, 1, 3, 2)
    s = _sc_weights(adj_t, W, aspect_onehot.astype(jnp.int32))
    return pl.pallas_call(
        _tc_body,
        in_specs=[
            pl.BlockSpec((B, N, IN_DIM), lambda: (0, 0, 0)),
            pl.BlockSpec((B, N, IN_DIM), lambda: (0, 0, 0)),
            pl.BlockSpec((1, IN_DIM + DEP + IN_DIM), lambda: (0, 0)),
        ],
        out_specs=pl.BlockSpec((B, N, IN_DIM), lambda: (0, 0, 0)),
        out_shape=jax.ShapeDtypeStruct((B, N, IN_DIM), jnp.float32),
    )(features, s, W)


# speedup vs baseline: 1.0669x; 1.0669x over previous
import functools
import jax
from jax import lax
import jax.numpy as jnp
from jax.experimental import pallas as pl
from jax.experimental.pallas import tpu as pltpu
from jax.experimental.pallas import tpu_sc as plsc

B, N, IN_DIM, DEP = 8, 64, 128, 8
K = N * DEP
ROWS = B * N
NC, NS, L = 2, 16, 16
NW = NC * NS
RPW = ROWS // NW
WPB = N // RPW
NCH = N // L

_mesh = plsc.VectorSubcoreMesh(core_axis_name="c", subcore_axis_name="s")


def _sc_body(adj_hbm, w_hbm, asp_hbm, s_hbm, adjv, sv, wv, av):
    wid = lax.axis_index("s") * NC + lax.axis_index("c")
    b = wid // WPB
    a0 = (wid % WPB) * RPW
    pltpu.sync_copy(adj_hbm.at[b, pl.ds(a0, RPW)], adjv)
    pltpu.sync_copy(w_hbm.at[0, pl.ds(IN_DIM, L)], wv)
    pltpu.sync_copy(asp_hbm.at[b, pl.ds(a0, RPW)], av)
    lanes = lax.iota(jnp.int32, L)
    lanesf = lanes.astype(jnp.float32)
    w2vec = wv[...]
    zeros = jnp.zeros((L,), jnp.float32)
    onehot = [(lanesf == float(d)).astype(jnp.float32) for d in range(DEP)]

    def row_body(row, carry):
        rowvec = jnp.broadcast_to(row, (L,))
        aspf = (plsc.load_gather(av, [rowvec]) != 0).astype(jnp.float32)
        cacc = [zeros for _ in range(NCH)]
        vacc = [zeros for _ in range(NCH)]
        radd = zeros
        for d in range(DEP):
            w2d = w2vec[d]
            rdi = jnp.zeros((L,), jnp.int32)
            for ch in range(NCH):
                g = adjv[row, d, pl.ds(L * ch, L)]
                m = jnp.abs(g) > 0.0
                cacc[ch] = cacc[ch] + jnp.where(m, 1.0, 0.0)
                vacc[ch] = vacc[ch] + g * w2d
                rdi = rdi + plsc.all_reduce_population_count(m)
            radd = radd + rdi.astype(jnp.float32) * onehot[d]
        cacc[0] = cacc[0] + radd
        for ch in range(NCH):
            sv[row, pl.ds(ch * L, L)] = (cacc[ch] * aspf) * jnp.exp(vacc[ch])
        for ch in range(NCH, IN_DIM // L):
            sv[row, pl.ds(ch * L, L)] = zeros
        return carry

    lax.fori_loop(0, RPW, row_body, 0)
    pltpu.sync_copy(sv, s_hbm.at[b, pl.ds(a0, RPW)])


_sc_weights = functools.partial(
    pl.kernel,
    out_type=jax.ShapeDtypeStruct((B, N, IN_DIM), jnp.float32),
    mesh=_mesh,
    compiler_params=pltpu.CompilerParams(
        needs_layout_passes=False, skip_device_barrier=True),
    scratch_types=[
        pltpu.VMEM((RPW, DEP, N), jnp.float32),
        pltpu.VMEM((RPW, IN_DIM), jnp.float32),
        pltpu.VMEM((L,), jnp.float32),
        pltpu.VMEM((RPW,), jnp.int32),
    ],
)(_sc_body)

_CONTRACT_LAST = (((1,), (1,)), ((), ()))


def _tc_body(f_ref, s_ref, w_ref, out_ref):
    w1row = w_ref[:, :IN_DIM]
    hi = jax.lax.Precision.HIGHEST
    f2 = f_ref[...].reshape(B * N, IN_DIM)
    eu = jnp.exp(lax.dot_general(w1row, f2, _CONTRACT_LAST, precision=hi))
    sts = [s_ref[b][:, :N] * eu[:, b * N:(b + 1) * N] for b in range(B)]
    nums = [jnp.dot(sts[b], f_ref[b], precision=hi) for b in range(B)]
    for b in range(B):
        den = jnp.sum(sts[b], axis=1, keepdims=True)
        out_ref[b] = jnp.where(den > 0.0, nums[b] / den, f_ref[b])


def kernel(features, aspect_onehot, adj_matrix, W):
    adj_t = adj_matrix.transpose(0, 1, 3, 2)
    s = _sc_weights(adj_t, W, aspect_onehot.astype(jnp.int32))
    return pl.pallas_call(
        _tc_body,
        in_specs=[
            pl.BlockSpec((B, N, IN_DIM), lambda: (0, 0, 0)),
            pl.BlockSpec((B, N, IN_DIM), lambda: (0, 0, 0)),
            pl.BlockSpec((1, IN_DIM + DEP + IN_DIM), lambda: (0, 0)),
        ],
        out_specs=pl.BlockSpec((B, N, IN_DIM), lambda: (0, 0, 0)),
        out_shape=jax.ShapeDtypeStruct((B, N, IN_DIM), jnp.float32),
    )(features, s, W)
